# Initial kernel scaffold; baseline (speedup 1.0000x reference)
import jax
import jax.numpy as jnp
from jax.experimental import pallas as pl


def kernel(x, edge_index, W, b):
    raise NotImplementedError("write your pallas kernel here")


# speedup vs baseline: 2.3851x; 2.3851x over previous
import functools

import jax
import jax.numpy as jnp
from jax import lax
from jax.experimental import pallas as pl
from jax.experimental.pallas import tpu as pltpu
from jax.experimental.pallas import tpu_sc as plsc

N_NODES = 10000
N_EDGES = 160000
D_IN = 256
D_OUT = 256
H = 128
NC = 2
NS = 16
EC = 128
CHUNKS_PER_SUB = 79
E_PAD = CHUNKS_PER_SUB * NS * EC
N_PAD = 10112
ROWS_PER_SUB = N_PAD // NS
ZROWS = 8
CROWS = 80


def _sc_segment_sum(x2, srcs, dst, oh):
    mesh = plsc.VectorSubcoreMesh(
        core_axis_name="c", subcore_axis_name="s",
        num_cores=NC, num_subcores=NS)

    @functools.partial(
        pl.kernel,
        mesh=mesh,
        out_type=(
            jax.ShapeDtypeStruct((NC, N_PAD, H), jnp.float32),
            jax.ShapeDtypeStruct((NC, CROWS, 128), jnp.float32),
        ),
        scratch_types=[
            pltpu.VMEM((EC,), jnp.int32),
            pltpu.VMEM((EC,), jnp.int32),
            pltpu.VMEM((EC, H), jnp.float32),
            pltpu.VMEM((ZROWS, H), jnp.float32),
            pltpu.VMEM_SHARED((N_PAD, H), jnp.float32),
            pltpu.VMEM_SHARED((CROWS, 128), jnp.float32),
            pltpu.SemaphoreType.DMA,
        ],
    )
    def sc_kernel(x2_hbm, srcs_hbm, dst_hbm, oh_hbm, s_hbm, cnt_hbm,
                  sidx, didx, rows, zb, acc, cnt, sem):
        cid = lax.axis_index("c")
        sid = lax.axis_index("s")
        zeros16 = jnp.zeros((16,), jnp.float32)

        def fill_zb(r, _):
            for j in range(H // 16):
                zb[r, pl.ds(j * 16, 16)] = zeros16
            return 0

        lax.fori_loop(0, ZROWS, fill_zb, 0)

        base = sid * ROWS_PER_SUB

        def zero_acc(k, _):
            pltpu.sync_copy(zb, acc.at[pl.ds(base + k * ZROWS, ZROWS)])
            return 0

        lax.fori_loop(0, ROWS_PER_SUB // ZROWS, zero_acc, 0)

        @pl.when(sid < CROWS // ZROWS)
        def _():
            pltpu.sync_copy(zb, cnt.at[pl.ds(sid * ZROWS, ZROWS)])

        plsc.subcore_barrier()

        def step(i, _):
            off = (sid * CHUNKS_PER_SUB + i) * EC
            pltpu.sync_copy(srcs_hbm.at[pl.ds(cid * E_PAD + off, EC)], sidx)
            pltpu.async_copy(x2_hbm.at[sidx], rows, sem).wait()
            pltpu.sync_copy(dst_hbm.at[pl.ds(off, EC)], didx)
            pltpu.sync_copy(rows, acc.at[didx], add=True)

            @pl.when(lax.rem(i, 2) == cid)
            def _():
                for j in range(EC // 16):
                    d16 = didx[pl.ds(j * 16, 16)]
                    sidx[pl.ds(j * 16, 16)] = lax.bitwise_and(d16, 127)
                    didx[pl.ds(j * 16, 16)] = lax.shift_right_logical(d16, 7)
                pltpu.async_copy(oh_hbm.at[sidx], rows, sem).wait()
                pltpu.sync_copy(rows, cnt.at[didx], add=True)

            return 0

        lax.fori_loop(0, CHUNKS_PER_SUB, step, 0)

        plsc.subcore_barrier()

        pltpu.sync_copy(acc.at[pl.ds(base, ROWS_PER_SUB)],
                        s_hbm.at[cid, pl.ds(base, ROWS_PER_SUB)])

        @pl.when(sid < CROWS // ZROWS)
        def _():
            pltpu.sync_copy(cnt.at[pl.ds(sid * ZROWS, ZROWS)],
                            cnt_hbm.at[cid, pl.ds(sid * ZROWS, ZROWS)])

    return sc_kernel(x2, srcs, dst, oh)


BR = 1000


def _tc_finish_body(x_ref, s0_ref, s1_ref, c0_ref, c1_ref, w_ref, b_ref,
                    o_ref):
    s = jnp.concatenate([s0_ref[0], s1_ref[0]], axis=1)
    c = jnp.maximum(c0_ref[...] + c1_ref[...], 1.0)
    h = s / c + x_ref[...]
    o_ref[...] = (
        jnp.dot(h, w_ref[...], preferred_element_type=jnp.float32,
                precision=lax.Precision.HIGHEST) + b_ref[...]
    )


def _tc_finish(x, s, c0, c1, W, b2):
    grid = (N_NODES // BR,)
    return pl.pallas_call(
        _tc_finish_body,
        grid=grid,
        in_specs=[
            pl.BlockSpec((BR, D_IN), lambda i: (i, 0)),
            pl.BlockSpec((1, BR, H), lambda i: (0, i, 0)),
            pl.BlockSpec((1, BR, H), lambda i: (1, i, 0)),
            pl.BlockSpec((BR, 1), lambda i: (i, 0)),
            pl.BlockSpec((BR, 1), lambda i: (i, 0)),
            pl.BlockSpec((D_IN, D_OUT), lambda i: (0, 0)),
            pl.BlockSpec((1, D_OUT), lambda i: (0, 0)),
        ],
        out_specs=pl.BlockSpec((BR, D_OUT), lambda i: (i, 0)),
        out_shape=jax.ShapeDtypeStruct((N_NODES, D_OUT), jnp.float32),
    )(x, s, s, c0, c1, W, b2)


def kernel(x, edge_index, W, b):
    src = edge_index[0].astype(jnp.int32)
    dst = edge_index[1].astype(jnp.int32)
    pad = E_PAD - N_EDGES
    srcs = jnp.concatenate([
        src, jnp.zeros((pad,), jnp.int32),
        src + N_NODES, jnp.full((pad,), N_NODES, jnp.int32),
    ])
    dst_p = jnp.concatenate([dst, jnp.full((pad,), N_NODES, jnp.int32)])
    x2 = jnp.concatenate([x[:, :H], x[:, H:]], axis=0)
    oh = jnp.eye(128, dtype=jnp.float32)

    s, cnt = _sc_segment_sum(x2, srcs, dst_p, oh)
    c0 = cnt[0].reshape(CROWS * 128, 1)[:N_NODES]
    c1 = cnt[1].reshape(CROWS * 128, 1)[:N_NODES]
    return _tc_finish(x, s, c0, c1, W, b.reshape(1, D_OUT))


# speedup vs baseline: 2.6174x; 1.0974x over previous
import functools

import jax
import jax.numpy as jnp
from jax import lax
from jax.experimental import pallas as pl
from jax.experimental.pallas import tpu as pltpu
from jax.experimental.pallas import tpu_sc as plsc

N_NODES = 10000
N_EDGES = 160000
D_IN = 256
D_OUT = 256
H = 128
NC = 2
NS = 16
EC = 128
CHUNKS_PER_SUB = 80
E_PAD = CHUNKS_PER_SUB * NS * EC
N_PAD = 10112
ROWS_PER_SUB = N_PAD // NS
ZROWS = 8
CROWS = 80


def _sc_segment_sum(x2, srcs4, dst3, oh):
    mesh = plsc.VectorSubcoreMesh(
        core_axis_name="c", subcore_axis_name="s",
        num_cores=NC, num_subcores=NS)

    @functools.partial(
        pl.kernel,
        mesh=mesh,
        out_type=(
            jax.ShapeDtypeStruct((NC, N_PAD, H), jnp.float32),
            jax.ShapeDtypeStruct((NC, CROWS, 128), jnp.float32),
        ),
        scratch_types=[
            pltpu.VMEM((CHUNKS_PER_SUB, EC), jnp.int32),
            pltpu.VMEM((2, 2, EC), jnp.int32),
            pltpu.VMEM((EC,), jnp.int32),
            pltpu.VMEM((EC,), jnp.int32),
            pltpu.VMEM((EC, H), jnp.float32),
            pltpu.VMEM((EC, H), jnp.float32),
            pltpu.VMEM((ZROWS, H), jnp.float32),
            pltpu.VMEM_SHARED((N_PAD, H), jnp.float32),
            pltpu.VMEM_SHARED((CROWS, 128), jnp.float32),
            pltpu.SemaphoreType.DMA,
            pltpu.SemaphoreType.DMA,
            pltpu.SemaphoreType.DMA,
        ],
    )
    def sc_kernel(x2_hbm, srcs_hbm, dst_hbm, oh_hbm, s_hbm, cnt_hbm,
                  sidx, didx, cl, ch, rows_a, rows_b, zb, acc, cnt,
                  sem_a, sem_b, sem_i):
        cid = lax.axis_index("c")
        sid = lax.axis_index("s")
        zeros16 = jnp.zeros((16,), jnp.float32)

        def fill_zb(r, _):
            for j in range(H // 16):
                zb[r, pl.ds(j * 16, 16)] = zeros16
            return 0

        lax.fori_loop(0, ZROWS, fill_zb, 0)

        base = sid * ROWS_PER_SUB

        def zero_acc(k, _):
            pltpu.sync_copy(zb, acc.at[pl.ds(base + k * ZROWS, ZROWS)])
            return 0

        lax.fori_loop(0, ROWS_PER_SUB // ZROWS, zero_acc, 0)

        @pl.when(sid < CROWS // ZROWS)
        def _():
            pltpu.sync_copy(zb, cnt.at[pl.ds(sid * ZROWS, ZROWS)])

        pltpu.sync_copy(srcs_hbm.at[cid, sid], sidx)
        pltpu.async_copy(dst_hbm.at[sid, pl.ds(0, 2)], didx.at[0], sem_i)

        plsc.subcore_barrier()

        def count_phase(kb, r, buf, sem):
            for j in range(EC // 16):
                d16 = didx[kb, r, pl.ds(j * 16, 16)]
                cl[pl.ds(j * 16, 16)] = lax.bitwise_and(d16, 127)
                ch[pl.ds(j * 16, 16)] = lax.shift_right_logical(d16, 7)
            pltpu.async_copy(oh_hbm.at[cl], buf, sem).wait()
            pltpu.sync_copy(buf, cnt.at[ch], add=True)

        pltpu.async_copy(x2_hbm.at[sidx.at[0]], rows_a, sem_a)

        def step(k, _):
            c0 = 2 * k
            kb = lax.rem(k, 2)
            pltpu.make_async_copy(
                dst_hbm.at[sid, pl.ds(0, 2)], didx.at[kb], sem_i).wait()

            @pl.when(k < CHUNKS_PER_SUB // 2 - 1)
            def _():
                pltpu.async_copy(dst_hbm.at[sid, pl.ds(c0 + 2, 2)],
                                 didx.at[1 - kb], sem_i)

            pltpu.async_copy(x2_hbm.at[sidx.at[c0 + 1]], rows_b, sem_b)
            pltpu.make_async_copy(x2_hbm.at[sidx.at[c0]], rows_a, sem_a).wait()
            pltpu.sync_copy(rows_a, acc.at[didx.at[kb, 0]], add=True)

            @pl.when(cid == 0)
            def _():
                count_phase(kb, 0, rows_a, sem_a)

            @pl.when(k < CHUNKS_PER_SUB // 2 - 1)
            def _():
                pltpu.async_copy(x2_hbm.at[sidx.at[c0 + 2]], rows_a, sem_a)

            pltpu.make_async_copy(
                x2_hbm.at[sidx.at[c0 + 1]], rows_b, sem_b).wait()
            pltpu.sync_copy(rows_b, acc.at[didx.at[kb, 1]], add=True)

            @pl.when(cid == 1)
            def _():
                count_phase(kb, 1, rows_b, sem_b)

            return 0

        lax.fori_loop(0, CHUNKS_PER_SUB // 2, step, 0)

        plsc.subcore_barrier()

        pltpu.sync_copy(acc.at[pl.ds(base, ROWS_PER_SUB)],
                        s_hbm.at[cid, pl.ds(base, ROWS_PER_SUB)])

        @pl.when(sid < CROWS // ZROWS)
        def _():
            pltpu.sync_copy(cnt.at[pl.ds(sid * ZROWS, ZROWS)],
                            cnt_hbm.at[cid, pl.ds(sid * ZROWS, ZROWS)])

    return sc_kernel(x2, srcs4, dst3, oh)


BR = 1000


def _tc_finish_body(x_ref, s0_ref, s1_ref, c0_ref, c1_ref, w_ref, b_ref,
                    o_ref):
    s = jnp.concatenate([s0_ref[0], s1_ref[0]], axis=1)
    c = jnp.maximum(c0_ref[...] + c1_ref[...], 1.0)
    h = s / c + x_ref[...]
    o_ref[...] = (
        jnp.dot(h, w_ref[...], preferred_element_type=jnp.float32,
                precision=lax.Precision.HIGHEST) + b_ref[...]
    )


def _tc_finish(x, s, c0, c1, W, b2):
    grid = (N_NODES // BR,)
    return pl.pallas_call(
        _tc_finish_body,
        grid=grid,
        in_specs=[
            pl.BlockSpec((BR, D_IN), lambda i: (i, 0)),
            pl.BlockSpec((1, BR, H), lambda i: (0, i, 0)),
            pl.BlockSpec((1, BR, H), lambda i: (1, i, 0)),
            pl.BlockSpec((BR, 1), lambda i: (i, 0)),
            pl.BlockSpec((BR, 1), lambda i: (i, 0)),
            pl.BlockSpec((D_IN, D_OUT), lambda i: (0, 0)),
            pl.BlockSpec((1, D_OUT), lambda i: (0, 0)),
        ],
        out_specs=pl.BlockSpec((BR, D_OUT), lambda i: (i, 0)),
        out_shape=jax.ShapeDtypeStruct((N_NODES, D_OUT), jnp.float32),
    )(x, s, s, c0, c1, W, b2)


def kernel(x, edge_index, W, b):
    src = edge_index[0].astype(jnp.int32)
    dst = edge_index[1].astype(jnp.int32)
    pad = E_PAD - N_EDGES
    srcs = jnp.concatenate([
        src, jnp.zeros((pad,), jnp.int32),
        src + N_NODES, jnp.full((pad,), N_NODES, jnp.int32),
    ]).reshape(NC, NS, CHUNKS_PER_SUB, EC)
    dst_p = jnp.concatenate(
        [dst, jnp.full((pad,), N_NODES, jnp.int32)]
    ).reshape(NS, CHUNKS_PER_SUB, EC)
    x2 = jnp.concatenate([x[:, :H], x[:, H:]], axis=0)
    oh = jnp.eye(128, dtype=jnp.float32)

    s, cnt = _sc_segment_sum(x2, srcs, dst_p, oh)
    c0 = cnt[0].reshape(CROWS * 128, 1)[:N_NODES]
    c1 = cnt[1].reshape(CROWS * 128, 1)[:N_NODES]
    return _tc_finish(x, s, c0, c1, W, b.reshape(1, D_OUT))
